# baseline (device time: 30823 ns/iter reference)
import jax
import jax.numpy as jnp
from jax import lax
from jax.experimental import pallas as pl
from jax.experimental.pallas import tpu as pltpu

N_DEV = 4


def kernel(x):
    _, m, n = x.shape
    half = m // 2
    qtr = m // 4
    blk = m // 8

    def body(x_ref, out_ref, stg_a, stg_b, recv_a1, recv_b1, recv_a2,
             recv_b2, acc_a, acc_b, g_a, g_b, ssem, rsem):
        p = lax.axis_index("i")
        q = p ^ 1
        r = 3 - p

        barrier_sem = pltpu.get_barrier_semaphore()
        for nbr in (q, r):
            pl.semaphore_signal(
                barrier_sem, inc=1,
                device_id=(nbr,), device_id_type=pl.DeviceIdType.MESH,
            )
        pl.semaphore_wait(barrier_sem, 2)

        j = jnp.where((p == 1) | (p == 2), 1, 0)
        k = p // 2
        jb = p // 2
        kb = p % 2
        fa_me = 2 * j + k
        fa_r = 2 * j + (1 - k)

        def rdma(src, dst, sem_idx, dev):
            return pltpu.make_async_remote_copy(
                src_ref=src, dst_ref=dst,
                send_sem=ssem.at[sem_idx], recv_sem=rsem.at[sem_idx],
                device_id=(dev,), device_id_type=pl.DeviceIdType.MESH,
            )

        def xbf16(row_start, nrows):
            return x_ref[0, pl.ds(row_start, nrows), :].astype(jnp.bfloat16)

        stg_a[...] = xbf16((1 - j) * qtr, qtr)
        a1 = rdma(stg_a, recv_a1, 0, q)
        a1.start()
        stg_b[...] = xbf16(half + (1 - jb) * qtr, qtr)
        b1 = rdma(stg_b, recv_b1, 1, r)
        b1.start()

        a1.wait()
        acc_a[...] = xbf16(j * qtr, qtr) + recv_a1[...]
        a2 = rdma(acc_a.at[pl.ds((1 - k) * blk, blk)], recv_a2, 2, r)
        a2.start()

        b1.wait()
        acc_b[...] = xbf16(half + jb * qtr, qtr) + recv_b1[...]
        b2 = rdma(acc_b.at[pl.ds((1 - kb) * blk, blk)], recv_b2, 3, q)
        b2.start()

        a2.wait()
        red_a = acc_a[pl.ds(k * blk, blk), :] + recv_a2[...]
        g_a[pl.ds(fa_me * blk, blk), :] = red_a
        a3 = rdma(g_a.at[pl.ds(fa_me * blk, blk)],
                  g_a.at[pl.ds(fa_me * blk, blk)], 4, r)
        a3.start()
        out_ref[pl.ds(fa_me * blk, blk), :] = red_a.astype(jnp.float32)

        b2.wait()
        red_b = acc_b[pl.ds(kb * blk, blk), :] + recv_b2[...]
        g_b[pl.ds(p * blk, blk), :] = red_b
        b3 = rdma(g_b.at[pl.ds(p * blk, blk)],
                  g_b.at[pl.ds(p * blk, blk)], 5, q)
        b3.start()
        out_ref[pl.ds(half + p * blk, blk), :] = red_b.astype(jnp.float32)

        a3.wait()
        a4 = rdma(g_a.at[pl.ds(j * qtr, qtr)],
                  g_a.at[pl.ds(j * qtr, qtr)], 6, q)
        a4.start()
        out_ref[pl.ds(fa_r * blk, blk), :] = (
            g_a[pl.ds(fa_r * blk, blk), :].astype(jnp.float32))

        b3.wait()
        b4 = rdma(g_b.at[pl.ds(jb * qtr, qtr)],
                  g_b.at[pl.ds(jb * qtr, qtr)], 7, r)
        b4.start()
        out_ref[pl.ds(half + q * blk, blk), :] = (
            g_b[pl.ds(q * blk, blk), :].astype(jnp.float32))

        a4.wait()
        out_ref[pl.ds((1 - j) * qtr, qtr), :] = (
            g_a[pl.ds((1 - j) * qtr, qtr), :].astype(jnp.float32))

        b4.wait()
        out_ref[pl.ds(half + (1 - jb) * qtr, qtr), :] = (
            g_b[pl.ds((1 - jb) * qtr, qtr), :].astype(jnp.float32))

    return pl.pallas_call(
        body,
        out_shape=jax.ShapeDtypeStruct((m, n), jnp.float32),
        in_specs=[pl.BlockSpec(memory_space=pltpu.VMEM)],
        out_specs=pl.BlockSpec(memory_space=pltpu.VMEM),
        scratch_shapes=[
            pltpu.VMEM((qtr, n), jnp.bfloat16),
            pltpu.VMEM((qtr, n), jnp.bfloat16),
            pltpu.VMEM((qtr, n), jnp.bfloat16),
            pltpu.VMEM((qtr, n), jnp.bfloat16),
            pltpu.VMEM((blk, n), jnp.bfloat16),
            pltpu.VMEM((blk, n), jnp.bfloat16),
            pltpu.VMEM((qtr, n), jnp.bfloat16),
            pltpu.VMEM((qtr, n), jnp.bfloat16),
            pltpu.VMEM((half, n), jnp.bfloat16),
            pltpu.VMEM((half, n), jnp.bfloat16),
            pltpu.SemaphoreType.DMA((8,)),
            pltpu.SemaphoreType.DMA((8,)),
        ],
        compiler_params=pltpu.CompilerParams(collective_id=0),
    )(x)


# device time: 29621 ns/iter; 1.0406x vs baseline; 1.0406x over previous
import jax
import jax.numpy as jnp
from jax import lax
from jax.experimental import pallas as pl
from jax.experimental.pallas import tpu as pltpu

N_DEV = 4


def kernel(x):
    _, m, n = x.shape
    half = m // 2
    qtr = m // 4

    def body(x_ref, out_ref, stg_a, stg_b, recv_a1, recv_b1, recv_a2,
             recv_b2, acc_a, acc_b, g_a, g_b, ssem, rsem):
        p = lax.axis_index("i")
        q = p ^ 1
        r = 3 - p

        barrier_sem = pltpu.get_barrier_semaphore()
        for nbr in (q, r):
            pl.semaphore_signal(
                barrier_sem, inc=1,
                device_id=(nbr,), device_id_type=pl.DeviceIdType.MESH,
            )
        pl.semaphore_wait(barrier_sem, 2)

        j = jnp.where((p == 1) | (p == 2), 1, 0)
        jb = p // 2

        def rdma(src, dst, sem_idx, dev):
            return pltpu.make_async_remote_copy(
                src_ref=src, dst_ref=dst,
                send_sem=ssem.at[sem_idx], recv_sem=rsem.at[sem_idx],
                device_id=(dev,), device_id_type=pl.DeviceIdType.MESH,
            )

        def xbf16(row_start):
            return x_ref[0, pl.ds(row_start, qtr), :].astype(jnp.bfloat16)

        stg_a[...] = xbf16((1 - j) * qtr)
        a1 = rdma(stg_a, recv_a1, 0, q)
        a1.start()
        stg_b[...] = xbf16(half + (1 - jb) * qtr)
        b1 = rdma(stg_b, recv_b1, 1, r)
        b1.start()

        a1.wait()
        acc_a[...] = xbf16(j * qtr) + recv_a1[...]
        a2 = rdma(acc_a, recv_a2, 2, r)
        a2.start()

        b1.wait()
        acc_b[...] = xbf16(half + jb * qtr) + recv_b1[...]
        b2 = rdma(acc_b, recv_b2, 3, q)
        b2.start()

        a2.wait()
        grp_a = acc_a[...] + recv_a2[...]
        g_a[pl.ds(j * qtr, qtr), :] = grp_a
        a3 = rdma(g_a.at[pl.ds(j * qtr, qtr)],
                  g_a.at[pl.ds(j * qtr, qtr)], 4, q)
        a3.start()
        out_ref[pl.ds(j * qtr, qtr), :] = grp_a.astype(jnp.float32)

        b2.wait()
        grp_b = acc_b[...] + recv_b2[...]
        g_b[pl.ds(jb * qtr, qtr), :] = grp_b
        b3 = rdma(g_b.at[pl.ds(jb * qtr, qtr)],
                  g_b.at[pl.ds(jb * qtr, qtr)], 5, r)
        b3.start()
        out_ref[pl.ds(half + jb * qtr, qtr), :] = grp_b.astype(jnp.float32)

        a3.wait()
        out_ref[pl.ds((1 - j) * qtr, qtr), :] = (
            g_a[pl.ds((1 - j) * qtr, qtr), :].astype(jnp.float32))

        b3.wait()
        out_ref[pl.ds(half + (1 - jb) * qtr, qtr), :] = (
            g_b[pl.ds((1 - jb) * qtr, qtr), :].astype(jnp.float32))

    return pl.pallas_call(
        body,
        out_shape=jax.ShapeDtypeStruct((m, n), jnp.float32),
        in_specs=[pl.BlockSpec(memory_space=pltpu.VMEM)],
        out_specs=pl.BlockSpec(memory_space=pltpu.VMEM),
        scratch_shapes=[
            pltpu.VMEM((qtr, n), jnp.bfloat16),
            pltpu.VMEM((qtr, n), jnp.bfloat16),
            pltpu.VMEM((qtr, n), jnp.bfloat16),
            pltpu.VMEM((qtr, n), jnp.bfloat16),
            pltpu.VMEM((qtr, n), jnp.bfloat16),
            pltpu.VMEM((qtr, n), jnp.bfloat16),
            pltpu.VMEM((qtr, n), jnp.bfloat16),
            pltpu.VMEM((qtr, n), jnp.bfloat16),
            pltpu.VMEM((half, n), jnp.bfloat16),
            pltpu.VMEM((half, n), jnp.bfloat16),
            pltpu.SemaphoreType.DMA((6,)),
            pltpu.SemaphoreType.DMA((6,)),
        ],
        compiler_params=pltpu.CompilerParams(collective_id=0),
    )(x)


# device time: 28852 ns/iter; 1.0683x vs baseline; 1.0267x over previous
import jax
import jax.numpy as jnp
from jax import lax
from jax.experimental import pallas as pl
from jax.experimental.pallas import tpu as pltpu

N_DEV = 4


def kernel(x):
    _, m, n = x.shape
    half = m // 2
    qtr = m // 4

    def body(x_ref, out_ref, stg_a, stg_b, recv_a1, recv_b1, recv_a2,
             recv_b2, acc_a, acc_b, ssem, rsem):
        p = lax.axis_index("i")
        q = p ^ 1
        r = 3 - p

        barrier_sem = pltpu.get_barrier_semaphore()
        for nbr in (q, r):
            pl.semaphore_signal(
                barrier_sem, inc=1,
                device_id=(nbr,), device_id_type=pl.DeviceIdType.MESH,
            )
        pl.semaphore_wait(barrier_sem, 2)

        j = jnp.where((p == 1) | (p == 2), 1, 0)
        jb = p // 2

        def rdma(src, dst, sem_idx, dev):
            return pltpu.make_async_remote_copy(
                src_ref=src, dst_ref=dst,
                send_sem=ssem.at[sem_idx], recv_sem=rsem.at[sem_idx],
                device_id=(dev,), device_id_type=pl.DeviceIdType.MESH,
            )

        def xbf16(row_start):
            return x_ref[0, pl.ds(row_start, qtr), :].astype(jnp.bfloat16)

        stg_a[...] = xbf16((1 - j) * qtr)
        a1 = rdma(stg_a, recv_a1, 0, q)
        a1.start()
        stg_b[...] = xbf16(half + (1 - jb) * qtr)
        b1 = rdma(stg_b, recv_b1, 1, r)
        b1.start()

        a1.wait()
        acc_a[...] = xbf16(j * qtr) + recv_a1[...]
        a2 = rdma(acc_a, recv_a2, 2, r)
        a2.start()

        b1.wait()
        acc_b[...] = xbf16(half + jb * qtr) + recv_b1[...]
        b2 = rdma(acc_b, recv_b2, 3, q)
        b2.start()

        a2.wait()
        out_ref[pl.ds(j * qtr, qtr), :] = acc_a[...] + recv_a2[...]
        a3 = rdma(out_ref.at[pl.ds(j * qtr, qtr)],
                  out_ref.at[pl.ds(j * qtr, qtr)], 4, q)
        a3.start()

        b2.wait()
        out_ref[pl.ds(half + jb * qtr, qtr), :] = acc_b[...] + recv_b2[...]
        b3 = rdma(out_ref.at[pl.ds(half + jb * qtr, qtr)],
                  out_ref.at[pl.ds(half + jb * qtr, qtr)], 5, r)
        b3.start()

        a3.wait()
        b3.wait()

    return pl.pallas_call(
        body,
        out_shape=jax.ShapeDtypeStruct((m, n), jnp.bfloat16),
        in_specs=[pl.BlockSpec(memory_space=pltpu.VMEM)],
        out_specs=pl.BlockSpec(memory_space=pltpu.VMEM),
        scratch_shapes=[
            pltpu.VMEM((qtr, n), jnp.bfloat16),
            pltpu.VMEM((qtr, n), jnp.bfloat16),
            pltpu.VMEM((qtr, n), jnp.bfloat16),
            pltpu.VMEM((qtr, n), jnp.bfloat16),
            pltpu.VMEM((qtr, n), jnp.bfloat16),
            pltpu.VMEM((qtr, n), jnp.bfloat16),
            pltpu.VMEM((qtr, n), jnp.bfloat16),
            pltpu.VMEM((qtr, n), jnp.bfloat16),
            pltpu.SemaphoreType.DMA((6,)),
            pltpu.SemaphoreType.DMA((6,)),
        ],
        compiler_params=pltpu.CompilerParams(collective_id=0),
    )(x)
